# baseline (speedup 1.0000x reference)
import jax
import jax.numpy as jnp
import numpy as np
from jax import lax
from jax.experimental import pallas as pl
from jax.experimental.pallas import tpu as pltpu

N_E = 8192
E_DIM = 256
V_PATCH = [1, 2, 3, 4, 5, 6, 8, 10, 13, 16]
QUANT_RESI = 0.5
BETA = 0.25
SHARE_K = 4
B = 16
H = 16
SN = len(V_PATCH)
N_TOK = B * H * H
PAD = 24
KB = 1024
CHUNK = 512


def _area_mat(in_s, out_s):
    M = np.zeros((out_s, in_s), np.float32)
    for i in range(out_s):
        s = (i * in_s) // out_s
        e = ((i + 1) * in_s + out_s - 1) // out_s
        M[i, s:e] = 1.0 / (e - s)
    return M


def _cub_w(x, a=-0.75):
    ax = abs(x)
    if ax <= 1.0:
        return (a + 2) * ax ** 3 - (a + 3) * ax ** 2 + 1.0
    if ax < 2.0:
        return a * ax ** 3 - 5 * a * ax ** 2 + 8 * a * ax - 4 * a
    return 0.0


def _up_mat(in_s, out_s):
    M = np.zeros((out_s, in_s), np.float32)
    for i in range(out_s):
        src = (i + 0.5) * in_s / out_s - 0.5
        fl = int(np.floor(src))
        for k in range(fl - 1, fl + 3):
            w = _cub_w(src - k)
            idx = min(max(k, 0), in_s - 1)
            M[i, idx] += w
    return M


def _pad8(n):
    return (n + 7) // 8 * 8


def _build_static():
    st = []
    for pn in V_PATCH[:-1]:
        pnsq = pn * pn
        P = _pad8(pnsq)
        M = _area_mat(H, pn)
        U = _up_mat(pn, H)
        d = {'pn': pn, 'P': P}
        if pn == 1:
            D0 = np.zeros((P, 256), np.float32)
            D0[0, :] = np.kron(M[0], M[0])
            d['dmats'] = [D0]
            U20 = np.zeros((256, P), np.float32)
            U20[:, 0] = np.kron(U[:, 0], U[:, 0])
            d['umats'] = [np.roll(U20, r, axis=0) for r in (0, -1, 1)]
        else:
            S1d = np.zeros((16 * pn, 256), np.float32)
            for w in range(16):
                for o in range(pn):
                    for h in range(16):
                        S1d[w * pn + o, h * 16 + w] = M[o, h]
            S2d = np.zeros((P, 16 * pn), np.float32)
            for o in range(pn):
                for p in range(pn):
                    for w in range(16):
                        S2d[o * pn + p, w * pn + o] = M[p, w]
            d['dmats'] = [S2d, S1d]
            S1u = np.zeros((16 * pn, P), np.float32)
            for w2 in range(pn):
                for o in range(16):
                    for h2 in range(pn):
                        S1u[w2 * 16 + o, h2 * pn + w2] = U[o, h2]
            S2u = np.zeros((256, 16 * pn), np.float32)
            for o in range(16):
                for p in range(16):
                    for w2 in range(pn):
                        S2u[o * 16 + p, w2 * 16 + o] = U[p, w2]
            d['umats'] = [np.roll(S2u, r, axis=0) for r in (0, -1, 1)] + [S1u]
        st.append(d)
    t = np.arange(N_TOK)
    y = (t % 256) // 16
    x = t % 16
    masks = np.zeros((N_TOK, 9), np.float32)
    for ky in range(3):
        for kx in range(3):
            ok = ((y + ky - 1 >= 0) & (y + ky - 1 < 16)
                  & (x + kx - 1 >= 0) & (x + kx - 1 < 16))
            masks[:, ky * 3 + kx] = ok.astype(np.float32)
    ticks = np.linspace(1.0 / 3 / SHARE_K, 1.0 - 1.0 / 3 / SHARE_K, SHARE_K)
    pis = [int(np.argmin(np.abs(ticks - si / (SN - 1)))) for si in range(SN)]
    return st, masks, pis


_ST, _MASKS, _PIS = _build_static()
_TPAD = [16 * d['P'] for d in _ST] + [N_TOK]
_NSM = [len(d['dmats']) + len(d['umats']) for d in _ST]


def _chunks(total, ch):
    out, c0 = [], 0
    while c0 < total:
        n = min(ch, total - c0)
        out.append((c0, n))
        c0 += n
    return out


def _fused_body(*refs):
    f32 = jnp.float32
    bf16 = jnp.bfloat16
    i = 0
    f_ref, embn_ref, e1_ref = refs[i:i + 3]; i += 3
    wts_ref, bias_ref, masks_ref = refs[i:i + 3]; i += 3
    smats = []
    for si in range(SN - 1):
        smats.append(refs[i:i + _NSM[si]]); i += _NSM[si]
    fhat_ref, sel_ref, loss_ref = refs[i:i + 3]; i += 3
    frest_ref, tv_ref, hc_ref, hl_ref, hr_ref, loss_sc = refs[i:i + 6]
    v_ref = tv_ref

    def init_body(c, _):
        s = pl.ds(c * CHUNK, CHUNK)
        frest_ref[s, :] = f_ref[s, :]
        fhat_ref[s, :] = jnp.zeros((CHUNK, E_DIM), f32)
        return 0
    lax.fori_loop(0, N_TOK // CHUNK, init_body, 0)
    zm = jnp.zeros((PAD, E_DIM), f32)
    for hp in (hc_ref, hl_ref, hr_ref):
        hp[0:PAD, :] = zm
        hp[PAD + N_TOK:PAD + N_TOK + PAD, :] = zm
    loss_sc[0] = jnp.float32(0.0)

    for si in range(SN):
        last = si == SN - 1
        if last:
            tsrc = frest_ref
        elif si == 0:
            d0 = smats[0][0]

            def down_body(b, _):
                rb = frest_ref[pl.ds(b * 256, 256), :]
                tv_ref[pl.ds(b * 8, 8), :] = jnp.dot(
                    d0[...], rb, preferred_element_type=f32)
                return 0
            lax.fori_loop(0, B, down_body, 0)
            tsrc = tv_ref
        else:
            s2d, s1d = smats[si][0], smats[si][1]
            P = _ST[si]['P']

            def down_body(b, _, s1d=s1d, s2d=s2d, P=P):
                rb = frest_ref[pl.ds(b * 256, 256), :]
                a1 = jnp.dot(s1d[...], rb, preferred_element_type=f32)
                tv_ref[pl.ds(b * P, P), :] = jnp.dot(
                    s2d[...], a1, preferred_element_type=f32)
                return 0
            lax.fori_loop(0, B, down_body, 0)
            tsrc = tv_ref

        T = _TPAD[si]
        for (c0, ch) in _chunks(T, CHUNK):
            tok = tsrc[c0:c0 + ch, :]
            nrm = jnp.sqrt(jnp.sum(tok * tok, axis=1, keepdims=True))
            tokn = (tok / jnp.maximum(nrm, 1e-12)).astype(bf16)

            def tile_body(k, carry, tokn=tokn, ch=ch, last=last):
                m, v, sel = carry
                et = embn_ref[:, pl.ds(k * KB, KB)]
                L = jnp.dot(tokn, et, preferred_element_type=f32)
                mk = jnp.max(L, axis=1, keepdims=True)
                iot = lax.broadcasted_iota(jnp.int32, (ch, KB), 1)
                cand = jnp.where(L == mk, iot, jnp.int32(1 << 30))
                ak = jnp.min(cand, axis=1, keepdims=True)
                ohf = (iot == ak).astype(f32)
                sl = pl.ds(k * KB, KB)
                vk = jnp.dot(ohf, e1_ref[sl, :], preferred_element_type=f32,
                             precision=lax.Precision.HIGHEST)
                upd = mk > m
                m = jnp.where(upd, mk, m)
                v = jnp.where(upd, vk, v)
                if last:
                    sel = jnp.where(upd, ak + k * KB, sel)
                return m, v, sel

            init = (jnp.full((ch, 1), -jnp.inf, f32),
                    jnp.zeros((ch, E_DIM), f32),
                    jnp.zeros((ch, 1), jnp.int32))
            _, v, sel = lax.fori_loop(0, N_E // KB, tile_body, init)
            if last:
                hc_ref[PAD + c0:PAD + c0 + ch, :] = v
                sel_ref[c0:c0 + ch, :] = sel
            else:
                v_ref[c0:c0 + ch, :] = v

        if last:
            for (c0, ch) in _chunks(N_TOK, CHUNK):
                hl_ref[PAD + c0:PAD + c0 + ch, :] = \
                    hc_ref[PAD + c0 + 1:PAD + c0 + ch + 1, :]
                hr_ref[PAD + c0:PAD + c0 + ch, :] = \
                    hc_ref[PAD + c0 - 1:PAD + c0 + ch - 1, :]
        elif si == 0:
            u2c, u2l, u2r = smats[0][1], smats[0][2], smats[0][3]

            def up_body(b, _):
                row = v_ref[pl.ds(b * 8, 1), :]
                dst = pl.ds(PAD + b * 256, 256)
                hc_ref[dst, :] = u2c[:, 0:1] * row
                hl_ref[dst, :] = u2l[:, 0:1] * row
                hr_ref[dst, :] = u2r[:, 0:1] * row
                return 0
            lax.fori_loop(0, B, up_body, 0)
        else:
            u2c, u2l, u2r, s1u = smats[si][2], smats[si][3], \
                smats[si][4], smats[si][5]
            P = _ST[si]['P']

            def up_body(b, _, u2c=u2c, u2l=u2l, u2r=u2r, s1u=s1u, P=P):
                vb = v_ref[pl.ds(b * P, P), :]
                t1 = jnp.dot(s1u[...], vb, preferred_element_type=f32)
                dst = pl.ds(PAD + b * 256, 256)
                hc_ref[dst, :] = jnp.dot(u2c[...], t1,
                                         preferred_element_type=f32)
                hl_ref[dst, :] = jnp.dot(u2l[...], t1,
                                         preferred_element_type=f32)
                hr_ref[dst, :] = jnp.dot(u2r[...], t1,
                                         preferred_element_type=f32)
                return 0
            lax.fori_loop(0, B, up_body, 0)

        pi = _PIS[si]
        loss_sc[1] = jnp.float32(0.0)

        def conv_body(c, _, pi=pi, last=last):
            c0 = c * CHUNK
            acc = None
            for ky in range(3):
                for kx in range(3):
                    tap = ky * 3 + kx
                    srcp = (hr_ref, hc_ref, hl_ref)[kx]
                    off = PAD + 16 * (ky - 1)
                    sh = srcp[pl.ds(c0 + off, CHUNK), :]
                    msk = masks_ref[pl.ds(c0, CHUNK), tap:tap + 1]
                    t = jnp.dot((sh * msk).astype(bf16), wts_ref[pi * 9 + tap],
                                preferred_element_type=f32)
                    acc = t if acc is None else acc + t
            conv = acc + jnp.broadcast_to(bias_ref[pi:pi + 1, :],
                                          (CHUNK, E_DIM))
            hcc = hc_ref[pl.ds(c0 + PAD, CHUNK), :]
            hnew = hcc * 0.5 + conv * 0.5
            s = pl.ds(c0, CHUNK)
            fh = fhat_ref[s, :] + hnew
            fhat_ref[s, :] = fh
            if not last:
                frest_ref[s, :] = frest_ref[s, :] - hnew
            d = fh - f_ref[s, :]
            loss_sc[1] += jnp.sum(d * d)
            return 0
        lax.fori_loop(0, N_TOK // CHUNK, conv_body, 0)
        m = loss_sc[1] * (1.0 / (N_TOK * E_DIM))
        loss_sc[0] += BETA * m + m

    loss_ref[:, :] = jnp.full((1, 1), loss_sc[0] * (1.0 / SN), f32)


def _enc_kernel(idx_ref, enc_ref, perp_ref, cnt_ref):
    i = pl.program_id(0)

    @pl.when(i == 0)
    def _():
        cnt_ref[:, :] = jnp.zeros((1, N_E), jnp.float32)

    iot = lax.broadcasted_iota(jnp.int32, (256, N_E), 1)
    oh = (iot == idx_ref[:, :]).astype(jnp.float32)
    enc_ref[:, :] = oh
    cnt_ref[:, :] += jnp.sum(oh, axis=0, keepdims=True)

    @pl.when(i == N_TOK // 256 - 1)
    def _():
        p = cnt_ref[:, :] * (1.0 / N_TOK)
        ent = jnp.sum(p * jnp.log(p + 1e-10))
        perp_ref[:, :] = jnp.full((1, 1), jnp.exp(-ent), jnp.float32)


@jax.jit
def kernel(f_BChw, embedding, phi_w, phi_b):
    f32 = jnp.float32
    bf16 = jnp.bfloat16
    f_tok = jnp.transpose(f_BChw, (0, 2, 3, 1)).reshape(N_TOK, E_DIM)
    emb_n = embedding / jnp.clip(
        jnp.linalg.norm(embedding, axis=1, keepdims=True), 1e-12)
    embn_T = emb_n.T.astype(bf16)
    wts = jnp.transpose(phi_w, (0, 3, 4, 2, 1)).reshape(
        SHARE_K * 9, E_DIM, E_DIM).astype(bf16)
    masks = jnp.asarray(_MASKS)
    smats = []
    for d in _ST:
        smats += [jnp.asarray(m) for m in d['dmats']]
        smats += [jnp.asarray(m) for m in d['umats']]

    fhat_tok, sel, loss = pl.pallas_call(
        _fused_body,
        out_shape=(
            jax.ShapeDtypeStruct((N_TOK, E_DIM), f32),
            jax.ShapeDtypeStruct((N_TOK, 1), jnp.int32),
            jax.ShapeDtypeStruct((1, 1), f32),
        ),
        scratch_shapes=[
            pltpu.VMEM((N_TOK, E_DIM), f32),
            pltpu.VMEM((B * _ST[-1]['P'], E_DIM), f32),
            pltpu.VMEM((N_TOK + 2 * PAD, E_DIM), f32),
            pltpu.VMEM((N_TOK + 2 * PAD, E_DIM), f32),
            pltpu.VMEM((N_TOK + 2 * PAD, E_DIM), f32),
            pltpu.SMEM((2,), f32),
        ],
        compiler_params=pltpu.CompilerParams(
            vmem_limit_bytes=100 * 1024 * 1024),
    )(f_tok, embn_T, embedding, wts, phi_b, masks, *smats)

    encodings, perp = pl.pallas_call(
        _enc_kernel,
        grid=(N_TOK // 256,),
        in_specs=[pl.BlockSpec((256, 1), lambda i: (i, 0))],
        out_specs=(pl.BlockSpec((256, N_E), lambda i: (i, 0)),
                   pl.BlockSpec((1, 1), lambda i: (0, 0))),
        out_shape=(jax.ShapeDtypeStruct((N_TOK, N_E), f32),
                   jax.ShapeDtypeStruct((1, 1), f32)),
        scratch_shapes=[pltpu.VMEM((1, N_E), f32)],
    )(sel)

    f_hat_out = fhat_tok.reshape(B, H, H, E_DIM).transpose(0, 3, 1, 2)
    last_idx = sel.reshape(N_TOK)
    return (f_hat_out, loss[0, 0], (perp[0, 0], encodings, last_idx))


# speedup vs baseline: 1.2911x; 1.2911x over previous
import jax
import jax.numpy as jnp
import numpy as np
from jax import lax
from jax.experimental import pallas as pl
from jax.experimental.pallas import tpu as pltpu

N_E = 8192
E_DIM = 256
V_PATCH = [1, 2, 3, 4, 5, 6, 8, 10, 13, 16]
QUANT_RESI = 0.5
BETA = 0.25
SHARE_K = 4
B = 16
H = 16
SN = len(V_PATCH)
N_TOK = B * H * H
PAD = 24
KB = 1024
CHUNK = 512


def _area_mat(in_s, out_s):
    M = np.zeros((out_s, in_s), np.float32)
    for i in range(out_s):
        s = (i * in_s) // out_s
        e = ((i + 1) * in_s + out_s - 1) // out_s
        M[i, s:e] = 1.0 / (e - s)
    return M


def _cub_w(x, a=-0.75):
    ax = abs(x)
    if ax <= 1.0:
        return (a + 2) * ax ** 3 - (a + 3) * ax ** 2 + 1.0
    if ax < 2.0:
        return a * ax ** 3 - 5 * a * ax ** 2 + 8 * a * ax - 4 * a
    return 0.0


def _up_mat(in_s, out_s):
    M = np.zeros((out_s, in_s), np.float32)
    for i in range(out_s):
        src = (i + 0.5) * in_s / out_s - 0.5
        fl = int(np.floor(src))
        for k in range(fl - 1, fl + 3):
            w = _cub_w(src - k)
            idx = min(max(k, 0), in_s - 1)
            M[i, idx] += w
    return M


def _pad8(n):
    return (n + 7) // 8 * 8


def _build_static():
    st = []
    for pn in V_PATCH[:-1]:
        pnsq = pn * pn
        P = _pad8(pnsq)
        M = _area_mat(H, pn)
        U = _up_mat(pn, H)
        d = {'pn': pn, 'P': P}
        if pn == 1:
            D0 = np.zeros((P, 256), np.float32)
            D0[0, :] = np.kron(M[0], M[0])
            d['dmats'] = [D0]
            U20 = np.zeros((256, P), np.float32)
            U20[:, 0] = np.kron(U[:, 0], U[:, 0])
            d['umats'] = [np.roll(U20, r, axis=0) for r in (0, -1, 1)]
        else:
            S1d = np.zeros((16 * pn, 256), np.float32)
            for w in range(16):
                for o in range(pn):
                    for h in range(16):
                        S1d[w * pn + o, h * 16 + w] = M[o, h]
            S2d = np.zeros((P, 16 * pn), np.float32)
            for o in range(pn):
                for p in range(pn):
                    for w in range(16):
                        S2d[o * pn + p, w * pn + o] = M[p, w]
            d['dmats'] = [S2d, S1d]
            S1u = np.zeros((16 * pn, P), np.float32)
            for w2 in range(pn):
                for o in range(16):
                    for h2 in range(pn):
                        S1u[w2 * 16 + o, h2 * pn + w2] = U[o, h2]
            S2u = np.zeros((256, 16 * pn), np.float32)
            for o in range(16):
                for p in range(16):
                    for w2 in range(pn):
                        S2u[o * 16 + p, w2 * 16 + o] = U[p, w2]
            d['umats'] = [np.roll(S2u, r, axis=0) for r in (0, -1, 1)] + [S1u]
        st.append(d)
    t = np.arange(N_TOK)
    y = (t % 256) // 16
    x = t % 16
    masks = np.zeros((N_TOK, 9), np.float32)
    for ky in range(3):
        for kx in range(3):
            ok = ((y + ky - 1 >= 0) & (y + ky - 1 < 16)
                  & (x + kx - 1 >= 0) & (x + kx - 1 < 16))
            masks[:, ky * 3 + kx] = ok.astype(np.float32)
    ticks = np.linspace(1.0 / 3 / SHARE_K, 1.0 - 1.0 / 3 / SHARE_K, SHARE_K)
    pis = [int(np.argmin(np.abs(ticks - si / (SN - 1)))) for si in range(SN)]
    return st, masks, pis


_ST, _MASKS, _PIS = _build_static()
_TPAD = [16 * d['P'] for d in _ST] + [N_TOK]


def _chunks(total, ch):
    out, c0 = [], 0
    while c0 < total:
        n = min(ch, total - c0)
        out.append((c0, n))
        c0 += n
    return out


def _recon_block(si, h_small_ref, umat_refs, wts_ref, bias_ref, masks_ref,
                 f_ref, frest_in, fhat_in, fr_out, fh_out, ssd_ref,
                 hc_ref, hl_ref, hr_ref, loss_sc):
    f32 = jnp.float32
    zm = jnp.zeros((PAD, E_DIM), f32)
    for hp in (hc_ref, hl_ref, hr_ref):
        hp[0:PAD, :] = zm
        hp[PAD + N_TOK:PAD + N_TOK + PAD, :] = zm
    if si == SN - 1:
        for (c0, ch) in _chunks(N_TOK, CHUNK):
            hc_ref[PAD + c0:PAD + c0 + ch, :] = h_small_ref[c0:c0 + ch, :]
        for (c0, ch) in _chunks(N_TOK, CHUNK):
            hl_ref[PAD + c0:PAD + c0 + ch, :] = \
                hc_ref[PAD + c0 + 1:PAD + c0 + ch + 1, :]
            hr_ref[PAD + c0:PAD + c0 + ch, :] = \
                hc_ref[PAD + c0 - 1:PAD + c0 + ch - 1, :]
    elif si == 0:
        u2c, u2l, u2r = umat_refs

        def up_body(b, _):
            row = h_small_ref[pl.ds(b * 8, 1), :]
            dst = pl.ds(PAD + b * 256, 256)
            hc_ref[dst, :] = u2c[:, 0:1] * row
            hl_ref[dst, :] = u2l[:, 0:1] * row
            hr_ref[dst, :] = u2r[:, 0:1] * row
            return 0
        lax.fori_loop(0, B, up_body, 0)
    else:
        u2c, u2l, u2r, s1u = umat_refs
        P = _ST[si]['P']

        def up_body(b, _):
            vb = h_small_ref[pl.ds(b * P, P), :]
            t1 = jnp.dot(s1u[...], vb, preferred_element_type=f32)
            dst = pl.ds(PAD + b * 256, 256)
            hc_ref[dst, :] = jnp.dot(u2c[...], t1, preferred_element_type=f32)
            hl_ref[dst, :] = jnp.dot(u2l[...], t1, preferred_element_type=f32)
            hr_ref[dst, :] = jnp.dot(u2r[...], t1, preferred_element_type=f32)
            return 0
        lax.fori_loop(0, B, up_body, 0)

    loss_sc[0] = jnp.float32(0.0)

    def conv_body(c, _):
        c0 = c * CHUNK
        acc = None
        for ky in range(3):
            for kx in range(3):
                tap = ky * 3 + kx
                srcp = (hr_ref, hc_ref, hl_ref)[kx]
                off = PAD + 16 * (ky - 1)
                sh = srcp[pl.ds(c0 + off, CHUNK), :]
                msk = masks_ref[pl.ds(c0, CHUNK), tap:tap + 1]
                t = jnp.dot(sh * msk, wts_ref[tap],
                            preferred_element_type=f32)
                acc = t if acc is None else acc + t
        conv = acc + jnp.broadcast_to(bias_ref[0:1, :], (CHUNK, E_DIM))
        hcc = hc_ref[pl.ds(c0 + PAD, CHUNK), :]
        hnew = hcc * 0.5 + conv * 0.5
        s = pl.ds(c0, CHUNK)
        fh = fhat_in[s, :] + hnew
        fh_out[s, :] = fh
        if fr_out is not None:
            fr_out[s, :] = frest_in[s, :] - hnew
        d = fh - f_ref[s, :]
        loss_sc[0] += jnp.sum(d * d)
        return 0
    lax.fori_loop(0, N_TOK // CHUNK, conv_body, 0)
    ssd_ref[:, :] = jnp.full((1, 1), loss_sc[0], f32)


def _search_block(si, src_ref, embn_ref, tok_ref, idx_ref, dmat_refs):
    f32 = jnp.float32
    if si == SN - 1:
        tsrc = src_ref
    elif si == 0:
        d0 = dmat_refs[0]

        def down_body(b, _):
            rb = src_ref[pl.ds(b * 256, 256), :]
            tok_ref[pl.ds(b * 8, 8), :] = jnp.dot(
                d0[...], rb, preferred_element_type=f32)
            return 0
        lax.fori_loop(0, B, down_body, 0)
        tsrc = tok_ref
    else:
        s2d, s1d = dmat_refs
        P = _ST[si]['P']

        def down_body(b, _):
            rb = src_ref[pl.ds(b * 256, 256), :]
            a1 = jnp.dot(s1d[...], rb, preferred_element_type=f32)
            tok_ref[pl.ds(b * P, P), :] = jnp.dot(
                s2d[...], a1, preferred_element_type=f32)
            return 0
        lax.fori_loop(0, B, down_body, 0)
        tsrc = tok_ref

    T = _TPAD[si]
    for (c0, ch) in _chunks(T, CHUNK):
        tok = tsrc[c0:c0 + ch, :]
        nrm = jnp.sqrt(jnp.sum(tok * tok, axis=1, keepdims=True))
        tokn = (tok / jnp.maximum(nrm, 1e-12)).astype(jnp.bfloat16)

        def tile_body(k, carry, tokn=tokn, ch=ch):
            m, sel = carry
            et = embn_ref[:, pl.ds(k * KB, KB)]
            L = jnp.dot(tokn, et, preferred_element_type=f32)
            mk = jnp.max(L, axis=1, keepdims=True)
            iot = lax.broadcasted_iota(jnp.int32, (ch, KB), 1)
            cand = jnp.where(L == mk, iot, jnp.int32(1 << 30))
            ak = jnp.min(cand, axis=1, keepdims=True)
            upd = mk > m
            return (jnp.where(upd, mk, m),
                    jnp.where(upd, ak + k * KB, sel))

        init = (jnp.full((ch, 1), -jnp.inf, f32),
                jnp.zeros((ch, 1), jnp.int32))
        _, sel = lax.fori_loop(0, N_E // KB, tile_body, init)
        idx_ref[c0:c0 + ch, :] = sel


def _make_stage(si):
    nu = 3 if si - 1 == 0 else 4
    nd = 1 if si == 0 else (0 if si == SN - 1 else 2)

    if si == 0:
        def body(*refs):
            (f_ref, embn_ref, d0_ref, idx_ref, tok_ref) = refs
            _search_block(0, f_ref, embn_ref, tok_ref, idx_ref, (d0_ref,))
        return body

    def body(*refs):
        i = 0
        frest_ref, fhat_ref, f_ref, hs_ref, embn_ref = refs[i:i + 5]; i += 5
        umats = refs[i:i + nu]; i += nu
        wts_ref, bias_ref, masks_ref = refs[i:i + 3]; i += 3
        dmats = refs[i:i + nd]; i += nd
        fr_out, fh_out, idx_ref, ssd_ref = refs[i:i + 4]; i += 4
        hc_ref, hl_ref, hr_ref, tok_ref, loss_sc = refs[i:i + 5]
        _recon_block(si - 1, hs_ref, umats, wts_ref, bias_ref, masks_ref,
                     f_ref, frest_ref, fhat_ref, fr_out, fh_out, ssd_ref,
                     hc_ref, hl_ref, hr_ref, loss_sc)
        _search_block(si, fr_out, embn_ref, tok_ref, idx_ref, dmats)
    return body


def _final_body(*refs):
    (fhat_ref, f_ref, hs_ref, wts_ref, bias_ref, masks_ref,
     fh_out, ssd_ref, hc_ref, hl_ref, hr_ref, loss_sc) = refs
    _recon_block(SN - 1, hs_ref, (), wts_ref, bias_ref, masks_ref,
                 f_ref, None, fhat_ref, None, fh_out, ssd_ref,
                 hc_ref, hl_ref, hr_ref, loss_sc)


def _enc_kernel(idx_ref, enc_ref, perp_ref, cnt_ref):
    i = pl.program_id(0)

    @pl.when(i == 0)
    def _():
        cnt_ref[:, :] = jnp.zeros((1, N_E), jnp.float32)

    iot = lax.broadcasted_iota(jnp.int32, (256, N_E), 1)
    oh = (iot == idx_ref[:, :]).astype(jnp.float32)
    enc_ref[:, :] = oh
    cnt_ref[:, :] += jnp.sum(oh, axis=0, keepdims=True)

    @pl.when(i == N_TOK // 256 - 1)
    def _():
        p = cnt_ref[:, :] * (1.0 / N_TOK)
        ent = jnp.sum(p * jnp.log(p + 1e-10))
        perp_ref[:, :] = jnp.full((1, 1), jnp.exp(-ent), jnp.float32)


_HPAD_SCR = pltpu.VMEM((N_TOK + 2 * PAD, E_DIM), jnp.float32)


@jax.jit
def kernel(f_BChw, embedding, phi_w, phi_b):
    f32 = jnp.float32
    f_tok = jnp.transpose(f_BChw, (0, 2, 3, 1)).reshape(N_TOK, E_DIM)
    emb_n = embedding / jnp.clip(
        jnp.linalg.norm(embedding, axis=1, keepdims=True), 1e-12)
    embn_T = emb_n.T.astype(jnp.bfloat16)
    wts = jnp.transpose(phi_w, (0, 3, 4, 2, 1)).reshape(SHARE_K * 9,
                                                        E_DIM, E_DIM)
    masks = jnp.asarray(_MASKS)
    sjnp = [{k: ([jnp.asarray(m) for m in d[k]] if isinstance(d[k], list)
                 else d[k]) for k in d} for d in _ST]

    idx0 = pl.pallas_call(
        _make_stage(0),
        out_shape=jax.ShapeDtypeStruct((_TPAD[0], 1), jnp.int32),
        scratch_shapes=[pltpu.VMEM((_TPAD[0], E_DIM), f32)],
    )(f_tok, embn_T, sjnp[0]['dmats'][0])

    f_rest = f_tok
    f_hat = jnp.zeros((N_TOK, E_DIM), f32)
    idx = idx0
    ssds = []
    for si in range(1, SN):
        prev = si - 1
        h_small = jnp.take(embedding, idx[:, 0], axis=0)
        pi = _PIS[prev]
        wts_p = wts[9 * pi:9 * pi + 9]
        bias_p = phi_b[pi:pi + 1, :]
        umats = sjnp[prev]['umats']
        dmats = sjnp[si]['dmats'] if si < SN - 1 else []
        f_rest, f_hat, idx, ssd = pl.pallas_call(
            _make_stage(si),
            out_shape=(
                jax.ShapeDtypeStruct((N_TOK, E_DIM), f32),
                jax.ShapeDtypeStruct((N_TOK, E_DIM), f32),
                jax.ShapeDtypeStruct((_TPAD[si], 1), jnp.int32),
                jax.ShapeDtypeStruct((1, 1), f32),
            ),
            scratch_shapes=[_HPAD_SCR, _HPAD_SCR, _HPAD_SCR,
                            pltpu.VMEM((_TPAD[si], E_DIM), f32),
                            pltpu.SMEM((1,), f32)],
            input_output_aliases={0: 0, 1: 1},
        )(f_rest, f_hat, f_tok, h_small, embn_T, *umats,
          wts_p, bias_p, masks, *dmats)
        ssds.append(ssd)

    h_small = jnp.take(embedding, idx[:, 0], axis=0)
    pi = _PIS[SN - 1]
    f_hat, ssd = pl.pallas_call(
        _final_body,
        out_shape=(jax.ShapeDtypeStruct((N_TOK, E_DIM), f32),
                   jax.ShapeDtypeStruct((1, 1), f32)),
        scratch_shapes=[_HPAD_SCR, _HPAD_SCR, _HPAD_SCR,
                        pltpu.SMEM((1,), f32)],
        input_output_aliases={0: 0},
    )(f_hat, f_tok, h_small, wts[9 * pi:9 * pi + 9],
      phi_b[pi:pi + 1, :], masks)
    ssds.append(ssd)

    loss = jnp.float32(0.0)
    for ssd in ssds:
        m = ssd[0, 0] * (1.0 / (N_TOK * E_DIM))
        loss = loss + BETA * m + m
    loss = loss / SN

    encodings, perp = pl.pallas_call(
        _enc_kernel,
        grid=(N_TOK // 256,),
        in_specs=[pl.BlockSpec((256, 1), lambda i: (i, 0))],
        out_specs=(pl.BlockSpec((256, N_E), lambda i: (i, 0)),
                   pl.BlockSpec((1, 1), lambda i: (0, 0))),
        out_shape=(jax.ShapeDtypeStruct((N_TOK, N_E), f32),
                   jax.ShapeDtypeStruct((1, 1), f32)),
        scratch_shapes=[pltpu.VMEM((1, N_E), f32)],
    )(idx)

    f_hat_out = f_hat.reshape(B, H, H, E_DIM).transpose(0, 3, 1, 2)
    last_idx = idx.reshape(N_TOK)
    return (f_hat_out, loss, (perp[0, 0], encodings, last_idx))
